# baseline (speedup 1.0000x reference)
import functools
import jax
import jax.numpy as jnp
from jax.experimental import pallas as pl


C = 64


def _edge_mlp_body(gd_ref, gs_ref, et_ref, w2_ref, b2_ref, m_ref):
    pre = gd_ref[...] + gs_ref[...] + et_ref[...]
    r = jnp.maximum(pre, 0.0)
    m_ref[...] = jnp.dot(r, w2_ref[...], preferred_element_type=jnp.float32) + b2_ref[...]


def _edge_mlp(gd, gs, et, W2, b2, eb=3200):
    E = gd.shape[0]
    grid = E // eb
    return pl.pallas_call(
        _edge_mlp_body,
        grid=(grid,),
        in_specs=[
            pl.BlockSpec((eb, C), lambda i: (i, 0)),
            pl.BlockSpec((eb, C), lambda i: (i, 0)),
            pl.BlockSpec((eb, C), lambda i: (i, 0)),
            pl.BlockSpec((C, C), lambda i: (0, 0)),
            pl.BlockSpec((1, C), lambda i: (0, 0)),
        ],
        out_specs=pl.BlockSpec((eb, C), lambda i: (i, 0)),
        out_shape=jax.ShapeDtypeStruct((E, C), jnp.float32),
    )(gd, gs, et, W2, b2.reshape(1, C))


def _readout_body(gs_ref, gd_ref, et_ref, w2_ref, q_ref):
    pre = gs_ref[...] + gd_ref[...] + et_ref[...]
    r = jnp.maximum(pre, 0.0)
    q_ref[...] = jnp.dot(r, w2_ref[...], preferred_element_type=jnp.float32)


def _readout(gs, gd, et, W2, eb=3200):
    E = gs.shape[0]
    grid = E // eb
    return pl.pallas_call(
        _readout_body,
        grid=(grid,),
        in_specs=[
            pl.BlockSpec((eb, C), lambda i: (i, 0)),
            pl.BlockSpec((eb, C), lambda i: (i, 0)),
            pl.BlockSpec((eb, C), lambda i: (i, 0)),
            pl.BlockSpec((C, 128), lambda i: (0, 0)),
        ],
        out_specs=pl.BlockSpec((eb, 128), lambda i: (i, 0)),
        out_shape=jax.ShapeDtypeStruct((E, 128), jnp.float32),
    )(gs, gd, et, W2)


def kernel(x, edge_index, edge_attr, W0, b0, Wm1, bm1, Wm2, bm2, Wu1, bu1, Wu2, bu2, Wr1, br1, Wr2, br2):
    n = x.shape[0]
    E = edge_index.shape[1]
    src = edge_index[0]
    dst = edge_index[1]

    cnt = jax.ops.segment_sum(jnp.ones((E,), jnp.float32), dst, num_segments=n)
    cnt_c = jnp.clip(cnt, 1.0)[:, None]
    mask = (cnt > 0)[:, None]
    et = edge_attr @ Wm1[2 * C:] + bm1
    rt = edge_attr @ Wr1[2 * C:] + br1

    h = x @ W0 + b0
    for _ in range(5):
        hr = jnp.maximum(h, 0.0)
        A = hr @ Wm1[:C]
        B = hr @ Wm1[C:2 * C]
        m = _edge_mlp(A[dst], B[src], et, Wm2, bm2)
        s = jax.ops.segment_sum(m, dst, num_segments=n)
        mean = s / cnt_c
        mean_sq = jax.ops.segment_sum(m * m, dst, num_segments=n) / cnt_c
        var = jnp.maximum(mean_sq - mean * mean, 0.0)
        std = jnp.where(mask, jnp.sqrt(var + 1e-5), 0.0)
        mn = jnp.where(mask, jax.ops.segment_min(m, dst, num_segments=n), 0.0)
        mx = jnp.where(mask, jax.ops.segment_max(m, dst, num_segments=n), 0.0)
        agg = jnp.concatenate([std, mn, mx, mean], axis=1)
        z = jnp.concatenate([agg, hr], axis=1)
        h = jnp.maximum(z @ Wu1 + bu1, 0.0) @ Wu2 + bu2

    P = h @ Wr1[:C]
    Q = h @ Wr1[C:2 * C]
    W2pad = jnp.zeros((C, 128), jnp.float32).at[:, :1].set(Wr2)
    qp = _readout(P[src], Q[dst], rt, W2pad)
    return qp[:, :1] + br2


# speedup vs baseline: 1.1489x; 1.1489x over previous
import functools
import jax
import jax.numpy as jnp
from jax import lax
from jax.experimental import pallas as pl
from jax.experimental.pallas import tpu as pltpu
from jax.experimental.pallas import tpu_sc as plsc


C = 64
N = 10000
NP = 10016
E = 320000
EP = 327680
NW = 32
NC = 2
EPW = EP // NW
KG = 512
NCH = EPW // KG

_sc_mesh = plsc.VectorSubcoreMesh(core_axis_name="c", subcore_axis_name="s")


def _sc_gather_body(a_hbm, b_hbm, et_hbm, dst_hbm, src_hbm, out_hbm,
                    dsti, srci, abuf, bbuf, ebuf, sem):
    w = lax.axis_index("s") * NC + lax.axis_index("c")
    base_w = w * EPW

    def chunk(ci, carry):
        base = base_w + ci * KG
        for j in range(4):
            pltpu.sync_copy(dst_hbm.at[pl.ds(base + j * 128, 128)], dsti.at[j])
            pltpu.sync_copy(src_hbm.at[pl.ds(base + j * 128, 128)], srci.at[j])
        cps = []
        for j in range(4):
            cps.append(pltpu.async_copy(
                a_hbm.at[dsti.at[j]], abuf.at[pl.ds(j * 128, 128)], sem))
            cps.append(pltpu.async_copy(
                b_hbm.at[srci.at[j]], bbuf.at[pl.ds(j * 128, 128)], sem))
        cps.append(pltpu.async_copy(et_hbm.at[pl.ds(base, KG)], ebuf, sem))
        for cp in cps:
            cp.wait()

        def row(r, carry2):
            for g in range(4):
                sl = pl.ds(g * 16, 16)
                v = abuf[r, sl] + bbuf[r, sl] + ebuf[r, sl]
                abuf[r, sl] = jnp.maximum(v, 0.0)
            return carry2

        lax.fori_loop(0, KG, row, 0, unroll=4)
        pltpu.sync_copy(abuf, out_hbm.at[pl.ds(base, KG)])
        return carry

    lax.fori_loop(0, NCH, chunk, 0)


_sc_gather = functools.partial(
    pl.kernel,
    out_type=jax.ShapeDtypeStruct((EP, C), jnp.float32),
    mesh=_sc_mesh,
    scratch_types=[
        pltpu.VMEM((4, 128), jnp.int32),
        pltpu.VMEM((4, 128), jnp.int32),
        pltpu.VMEM((KG, C), jnp.float32),
        pltpu.VMEM((KG, C), jnp.float32),
        pltpu.VMEM((KG, C), jnp.float32),
        pltpu.SemaphoreType.DMA,
    ],
    compiler_params=pltpu.CompilerParams(use_tc_tiling_on_sc=False),
)(_sc_gather_body)


def _node_prep_body(h_ref, wd_ref, ws_ref, hr_ref, a_ref, b_ref):
    hr = jnp.maximum(h_ref[...], 0.0)
    hr_ref[...] = hr
    a_ref[...] = jnp.dot(hr, wd_ref[...], preferred_element_type=jnp.float32)
    b_ref[...] = jnp.dot(hr, ws_ref[...], preferred_element_type=jnp.float32)


def _node_prep(h, Wd, Ws):
    return pl.pallas_call(
        _node_prep_body,
        out_shape=(
            jax.ShapeDtypeStruct((NP, C), jnp.float32),
            jax.ShapeDtypeStruct((NP, C), jnp.float32),
            jax.ShapeDtypeStruct((NP, C), jnp.float32),
        ),
    )(h, Wd, Ws)


def _readout_prep_body(h_ref, wd_ref, ws_ref, p_ref, q_ref):
    h = h_ref[...]
    p_ref[...] = jnp.dot(h, wd_ref[...], preferred_element_type=jnp.float32)
    q_ref[...] = jnp.dot(h, ws_ref[...], preferred_element_type=jnp.float32)


def _readout_prep(h, Wsrc, Wdst):
    return pl.pallas_call(
        _readout_prep_body,
        out_shape=(
            jax.ShapeDtypeStruct((NP, C), jnp.float32),
            jax.ShapeDtypeStruct((NP, C), jnp.float32),
        ),
    )(h, Wsrc, Wdst)


def _edge_terms_body(ea_ref, wme_ref, bm_ref, wre_ref, br_ref, et_ref, rt_ref):
    ea = ea_ref[...]
    et_ref[...] = jnp.dot(ea, wme_ref[...], preferred_element_type=jnp.float32) + bm_ref[...]
    rt_ref[...] = jnp.dot(ea, wre_ref[...], preferred_element_type=jnp.float32) + br_ref[...]


def _edge_terms(ea8, Wme8, bm1, Wre8, br1, eb=4096):
    grid = EP // eb
    return pl.pallas_call(
        _edge_terms_body,
        grid=(grid,),
        in_specs=[
            pl.BlockSpec((eb, 8), lambda i: (i, 0)),
            pl.BlockSpec((8, C), lambda i: (0, 0)),
            pl.BlockSpec((1, C), lambda i: (0, 0)),
            pl.BlockSpec((8, C), lambda i: (0, 0)),
            pl.BlockSpec((1, C), lambda i: (0, 0)),
        ],
        out_specs=(
            pl.BlockSpec((eb, C), lambda i: (i, 0)),
            pl.BlockSpec((eb, C), lambda i: (i, 0)),
        ),
        out_shape=(
            jax.ShapeDtypeStruct((EP, C), jnp.float32),
            jax.ShapeDtypeStruct((EP, C), jnp.float32),
        ),
    )(ea8, Wme8, bm1.reshape(1, C), Wre8, br1.reshape(1, C))


def _h0_body(x_ref, w_ref, b_ref, h_ref):
    h_ref[...] = jnp.dot(x_ref[...], w_ref[...], preferred_element_type=jnp.float32) + b_ref[...]


def _h0(x128, W0128, b0):
    return pl.pallas_call(
        _h0_body,
        out_shape=jax.ShapeDtypeStruct((NP, C), jnp.float32),
    )(x128, W0128, b0.reshape(1, C))


def _mm_bias_body(r_ref, w_ref, b_ref, m_ref):
    m_ref[...] = jnp.dot(r_ref[...], w_ref[...], preferred_element_type=jnp.float32) + b_ref[...]


def _edge_mlp2(r, W2, b2, eb=4096):
    grid = EP // eb
    return pl.pallas_call(
        _mm_bias_body,
        grid=(grid,),
        in_specs=[
            pl.BlockSpec((eb, C), lambda i: (i, 0)),
            pl.BlockSpec((C, C), lambda i: (0, 0)),
            pl.BlockSpec((1, C), lambda i: (0, 0)),
        ],
        out_specs=pl.BlockSpec((eb, C), lambda i: (i, 0)),
        out_shape=jax.ShapeDtypeStruct((EP, C), jnp.float32),
    )(r, W2, b2.reshape(1, C))


def _readout2_body(t_ref, w_ref, q_ref):
    q_ref[...] = jnp.dot(t_ref[...], w_ref[...], preferred_element_type=jnp.float32)


def _readout2(t, W2pad, eb=4096):
    grid = EP // eb
    return pl.pallas_call(
        _readout2_body,
        grid=(grid,),
        in_specs=[
            pl.BlockSpec((eb, C), lambda i: (i, 0)),
            pl.BlockSpec((C, 128), lambda i: (0, 0)),
        ],
        out_specs=pl.BlockSpec((eb, 128), lambda i: (i, 0)),
        out_shape=jax.ShapeDtypeStruct((EP, 128), jnp.float32),
    )(t, W2pad)


def _update_body(s_ref, q_ref, mn_ref, mx_ref, hr_ref, inv_ref, msk_ref,
                 w1_ref, b1_ref, w2_ref, b2_ref, h_ref):
    inv = inv_ref[...]
    msk = msk_ref[...]
    mean = s_ref[...] * inv
    mean_sq = q_ref[...] * inv
    var = jnp.maximum(mean_sq - mean * mean, 0.0)
    std = jnp.sqrt(var + 1e-5) * msk
    mn = mn_ref[...] * msk
    mx = mx_ref[...] * msk
    z = jnp.concatenate([std, mn, mx, mean, hr_ref[...]], axis=1)
    u = jnp.maximum(jnp.dot(z, w1_ref[...], preferred_element_type=jnp.float32) + b1_ref[...], 0.0)
    h_ref[...] = jnp.dot(u, w2_ref[...], preferred_element_type=jnp.float32) + b2_ref[...]


def _update(s, q, mn, mx, hr, inv, msk, Wu1, bu1, Wu2, bu2):
    return pl.pallas_call(
        _update_body,
        out_shape=jax.ShapeDtypeStruct((NP, C), jnp.float32),
    )(s, q, mn, mx, hr, inv, msk, Wu1, bu1.reshape(1, C), Wu2, bu2.reshape(1, C))


def kernel(x, edge_index, edge_attr, W0, b0, Wm1, bm1, Wm2, bm2, Wu1, bu1, Wu2, bu2, Wr1, br1, Wr2, br2):
    src = edge_index[0]
    dst = edge_index[1]
    src_p = jnp.pad(src, (0, EP - E)).astype(jnp.int32)
    dst_p = jnp.pad(dst, (0, EP - E)).astype(jnp.int32)
    ea8 = jnp.pad(edge_attr, ((0, EP - E), (0, 6)))

    Wme8 = jnp.pad(Wm1[2 * C:], ((0, 6), (0, 0)))
    Wre8 = jnp.pad(Wr1[2 * C:], ((0, 6), (0, 0)))
    x128 = jnp.pad(x, ((0, NP - N), (0, 125)))
    W0128 = jnp.pad(W0, ((0, 125), (0, 0)))

    et, rt = _edge_terms(ea8, Wme8, bm1, Wre8, br1)
    h = _h0(x128, W0128, b0)

    cnt = jax.ops.segment_sum(jnp.ones((E,), jnp.float32), dst, num_segments=N)
    cntp = jnp.pad(cnt, (0, NP - N))
    inv = (1.0 / jnp.clip(cntp, 1.0))[:, None]
    msk = (cntp > 0).astype(jnp.float32)[:, None]

    for _ in range(5):
        hr, A, B = _node_prep(h, Wm1[:C], Wm1[C:2 * C])
        r = _sc_gather(A, B, et, dst_p, src_p)
        m = _edge_mlp2(r, Wm2, bm2)
        mv = m[:E]
        s = jnp.pad(jax.ops.segment_sum(mv, dst, num_segments=N), ((0, NP - N), (0, 0)))
        q = jnp.pad(jax.ops.segment_sum(mv * mv, dst, num_segments=N), ((0, NP - N), (0, 0)))
        mn = jnp.pad(jax.ops.segment_min(mv, dst, num_segments=N), ((0, NP - N), (0, 0)))
        mx = jnp.pad(jax.ops.segment_max(mv, dst, num_segments=N), ((0, NP - N), (0, 0)))
        h = _update(s, q, mn, mx, hr, inv, msk, Wu1, bu1, Wu2, bu2)

    P, Q = _readout_prep(h, Wr1[:C], Wr1[C:2 * C])
    t = _sc_gather(Q, P, rt, dst_p, src_p)
    W2pad = jnp.zeros((C, 128), jnp.float32).at[:, :1].set(Wr2)
    qp = _readout2(t, W2pad)
    return qp[:E, :1] + br2


# speedup vs baseline: 2.1762x; 1.8942x over previous
import functools
import jax
import jax.numpy as jnp
from jax import lax
from jax.experimental import pallas as pl
from jax.experimental.pallas import tpu as pltpu
from jax.experimental.pallas import tpu_sc as plsc


C = 64
N = 10000
NP = 10016
E = 320000
EP = 327680
NW = 32
NC = 2
EPW = EP // NW
KG = 512
NCH = EPW // KG

_sc_mesh = plsc.VectorSubcoreMesh(core_axis_name="c", subcore_axis_name="s")


def _sc_gather_body(a_hbm, b_hbm, et_hbm, dst_hbm, src_hbm, out_hbm,
                    dsti, srci, abuf, bbuf, ebuf, sem):
    w = lax.axis_index("s") * NC + lax.axis_index("c")
    base_w = w * EPW

    def chunk(ci, carry):
        base = base_w + ci * KG
        for j in range(4):
            pltpu.sync_copy(dst_hbm.at[pl.ds(base + j * 128, 128)], dsti.at[j])
            pltpu.sync_copy(src_hbm.at[pl.ds(base + j * 128, 128)], srci.at[j])
        cps = []
        for j in range(4):
            cps.append(pltpu.async_copy(
                a_hbm.at[dsti.at[j]], abuf.at[pl.ds(j * 128, 128)], sem))
            cps.append(pltpu.async_copy(
                b_hbm.at[srci.at[j]], bbuf.at[pl.ds(j * 128, 128)], sem))
        cps.append(pltpu.async_copy(et_hbm.at[pl.ds(base, KG)], ebuf, sem))
        for cp in cps:
            cp.wait()

        def row(r, carry2):
            for g in range(4):
                sl = pl.ds(g * 16, 16)
                v = abuf[r, sl] + bbuf[r, sl] + ebuf[r, sl]
                abuf[r, sl] = jnp.maximum(v, 0.0)
            return carry2

        lax.fori_loop(0, KG, row, 0, unroll=4)
        pltpu.sync_copy(abuf, out_hbm.at[pl.ds(base, KG)])
        return carry

    lax.fori_loop(0, NCH, chunk, 0)


_sc_gather = functools.partial(
    pl.kernel,
    out_type=jax.ShapeDtypeStruct((EP, C), jnp.float32),
    mesh=_sc_mesh,
    scratch_types=[
        pltpu.VMEM((4, 128), jnp.int32),
        pltpu.VMEM((4, 128), jnp.int32),
        pltpu.VMEM((KG, C), jnp.float32),
        pltpu.VMEM((KG, C), jnp.float32),
        pltpu.VMEM((KG, C), jnp.float32),
        pltpu.SemaphoreType.DMA,
    ],
    compiler_params=pltpu.CompilerParams(use_tc_tiling_on_sc=False, needs_layout_passes=False),
)(_sc_gather_body)


NB = 10016
NPT = 313


def _iota16():
    return lax.broadcasted_iota(jnp.int32, (16,), 0)


def _sc_hist_body(dst_hbm, out_hbm, hist, dbuf, sem):
    w = lax.axis_index("s") * NC + lax.axis_index("c")
    ones = jnp.ones((16,), jnp.int32)

    def zero(i, carry):
        hist[pl.ds(i * 16, 16)] = jnp.zeros((16,), jnp.int32)
        return carry

    lax.fori_loop(0, NB // 16, zero, 0)

    def chunk(ci, carry):
        base = w * EPW + ci * KG
        pltpu.sync_copy(dst_hbm.at[pl.ds(base, KG)], dbuf)

        def group(g, c2):
            v = dbuf[pl.ds(g * 16, 16)]
            plsc.addupdate_scatter(hist, [v], ones)
            return c2

        lax.fori_loop(0, KG // 16, group, 0)
        return carry

    lax.fori_loop(0, NCH, chunk, 0)
    pltpu.sync_copy(hist, out_hbm.at[w])


_sc_hist = functools.partial(
    pl.kernel,
    out_type=jax.ShapeDtypeStruct((NW, NB), jnp.int32),
    mesh=_sc_mesh,
    scratch_types=[
        pltpu.VMEM((NB,), jnp.int32),
        pltpu.VMEM((KG,), jnp.int32),
        pltpu.SemaphoreType.DMA,
    ],
    compiler_params=pltpu.CompilerParams(use_tc_tiling_on_sc=False, needs_layout_passes=False),
)(_sc_hist_body)


def _lane_gather(v, idx):
    return lax.gather(
        v, idx[:, None],
        dimension_numbers=lax.GatherDimensionNumbers(
            offset_dims=(), collapsed_slice_dims=(0,), start_index_map=(0,)),
        slice_sizes=(1,),
        mode=lax.GatherScatterMode.PROMISE_IN_BOUNDS)


def _dup_rank(v):
    iota = _iota16()
    rank = jnp.zeros((16,), jnp.int32)
    for k in range(1, 16):
        shifted = _lane_gather(v, jnp.clip(iota - k, 0, 15))
        eq = jnp.logical_and(shifted == v, iota >= k)
        rank = rank + eq.astype(jnp.int32)
    return rank


def _sc_place_body(dst_hbm, src_hbm, ea_hbm, offs_hbm,
                   dsts_hbm, srcs_hbm, eas_hbm,
                   ctr, dstb, srcb, eab, posb, sem):
    w = lax.axis_index("s") * NC + lax.axis_index("c")
    ones = jnp.ones((16,), jnp.int32)
    pltpu.sync_copy(offs_hbm.at[w], ctr)

    def chunk(ci, carry):
        base = w * EPW + ci * KG
        for j in range(4):
            pltpu.sync_copy(dst_hbm.at[pl.ds(base + j * 128, 128)], dstb.at[j])
            pltpu.sync_copy(src_hbm.at[pl.ds(base + j * 128, 128)], srcb.at[j])
            pltpu.sync_copy(ea_hbm.at[pl.ds(base + j * 128, 128)], eab.at[j])
        for j in range(4):
            for g in range(8):
                v = dstb[j, pl.ds(g * 16, 16)]
                old = plsc.load_gather(ctr, [v])
                rank = _dup_rank(v)
                plsc.addupdate_scatter(ctr, [v], ones)
                posb[j, pl.ds(g * 16, 16)] = old + rank
        cps = []
        for j in range(4):
            cps.append(pltpu.async_copy(dstb.at[j], dsts_hbm.at[posb.at[j]], sem))
            cps.append(pltpu.async_copy(srcb.at[j], srcs_hbm.at[posb.at[j]], sem))
            cps.append(pltpu.async_copy(eab.at[j], eas_hbm.at[posb.at[j]], sem))
        for cp in cps:
            cp.wait()
        return carry

    lax.fori_loop(0, NCH, chunk, 0)


_sc_place = functools.partial(
    pl.kernel,
    out_type=(
        jax.ShapeDtypeStruct((EP,), jnp.int32),
        jax.ShapeDtypeStruct((EP,), jnp.int32),
        jax.ShapeDtypeStruct((EP, 8), jnp.float32),
    ),
    mesh=_sc_mesh,
    scratch_types=[
        pltpu.VMEM((NB,), jnp.int32),
        pltpu.VMEM((4, 128), jnp.int32),
        pltpu.VMEM((4, 128), jnp.int32),
        pltpu.VMEM((4, 128, 8), jnp.float32),
        pltpu.VMEM((4, 128), jnp.int32),
        pltpu.SemaphoreType.DMA,
    ],
    compiler_params=pltpu.CompilerParams(use_tc_tiling_on_sc=False, needs_layout_passes=False),
)(_sc_place_body)


K2 = 256
ACC = NPT * C


def _sc_scatter_body(m_hbm, dsts_hbm, rp_hbm,
                     aggs_hbm, aggq_hbm, aggn_hbm, aggx_hbm,
                     accs, accq, accn, accx, mbuf, dbuf, rpb, sem):
    w = lax.axis_index("s") * NC + lax.axis_index("c")
    nbase = w * NPT
    iota = _iota16()

    def fetch(idx):
        b16 = (idx // 16) * 16
        pltpu.sync_copy(rp_hbm.at[pl.ds(b16, 16)], rpb)
        v = rpb[...]
        return jnp.sum(jnp.where(iota == (idx - b16), v, 0))

    start = fetch(nbase)
    end = fetch(nbase + NPT)

    def zero(i, carry):
        sl = pl.ds(i * 16, 16)
        accs[sl] = jnp.zeros((16,), jnp.float32)
        accq[sl] = jnp.zeros((16,), jnp.float32)
        accn[sl] = jnp.full((16,), 1e30, jnp.float32)
        accx[sl] = jnp.full((16,), -1e30, jnp.float32)
        return carry

    lax.fori_loop(0, ACC // 16, zero, 0)

    start16 = (start // 16) * 16
    nch = (end - start16 + K2 - 1) // K2

    def chunk(ci, carry):
        base = start16 + ci * K2
        pltpu.sync_copy(m_hbm.at[pl.ds(base, K2)], mbuf)
        pltpu.sync_copy(dsts_hbm.at[pl.ds(base, K2)], dbuf)

        def group(g, c2):
            dv = dbuf[pl.ds(g * 16, 16)]
            posv = base + g * 16 + iota
            validv = jnp.logical_and(posv >= start, posv < end)
            locc = jnp.clip(dv - nbase, 0, NPT - 1)
            for e in range(16):
                sel = iota == e
                row = jnp.sum(jnp.where(sel, locc, 0))
                vald = jnp.sum(jnp.where(sel, validv.astype(jnp.int32), 0))
                vb = jnp.full((16,), vald, jnp.int32) > 0
                vf = jnp.where(vb, 1.0, 0.0)
                off = row * C
                for grp in range(4):
                    mv = mbuf[g * 16 + e, pl.ds(grp * 16, 16)]
                    sl = pl.ds(off + grp * 16, 16)
                    accs[sl] = accs[sl] + mv * vf
                    accq[sl] = accq[sl] + mv * mv * vf
                    accn[sl] = jnp.minimum(accn[sl], jnp.where(vb, mv, 1e30))
                    accx[sl] = jnp.maximum(accx[sl], jnp.where(vb, mv, -1e30))
            return c2

        lax.fori_loop(0, K2 // 16, group, 0)
        return carry

    lax.fori_loop(0, nch, chunk, 0)
    obase = w * ACC
    pltpu.sync_copy(accs, aggs_hbm.at[pl.ds(obase, ACC)])
    pltpu.sync_copy(accq, aggq_hbm.at[pl.ds(obase, ACC)])
    pltpu.sync_copy(accn, aggn_hbm.at[pl.ds(obase, ACC)])
    pltpu.sync_copy(accx, aggx_hbm.at[pl.ds(obase, ACC)])


_sc_scatter = functools.partial(
    pl.kernel,
    out_type=(
        jax.ShapeDtypeStruct((NP * C,), jnp.float32),
        jax.ShapeDtypeStruct((NP * C,), jnp.float32),
        jax.ShapeDtypeStruct((NP * C,), jnp.float32),
        jax.ShapeDtypeStruct((NP * C,), jnp.float32),
    ),
    mesh=_sc_mesh,
    scratch_types=[
        pltpu.VMEM((ACC,), jnp.float32),
        pltpu.VMEM((ACC,), jnp.float32),
        pltpu.VMEM((ACC,), jnp.float32),
        pltpu.VMEM((ACC,), jnp.float32),
        pltpu.VMEM((K2, C), jnp.float32),
        pltpu.VMEM((K2,), jnp.int32),
        pltpu.VMEM((16,), jnp.int32),
        pltpu.SemaphoreType.DMA,
    ],
    compiler_params=pltpu.CompilerParams(use_tc_tiling_on_sc=False, needs_layout_passes=False),
)(_sc_scatter_body)


def _node_prep_body(h_ref, wd_ref, ws_ref, hr_ref, a_ref, b_ref):
    hr = jnp.maximum(h_ref[...], 0.0)
    hr_ref[...] = hr
    a_ref[...] = jnp.dot(hr, wd_ref[...], preferred_element_type=jnp.float32)
    b_ref[...] = jnp.dot(hr, ws_ref[...], preferred_element_type=jnp.float32)


def _node_prep(h, Wd, Ws):
    return pl.pallas_call(
        _node_prep_body,
        out_shape=(
            jax.ShapeDtypeStruct((NP, C), jnp.float32),
            jax.ShapeDtypeStruct((NP, C), jnp.float32),
            jax.ShapeDtypeStruct((NP, C), jnp.float32),
        ),
    )(h, Wd, Ws)


def _readout_prep_body(h_ref, wd_ref, ws_ref, p_ref, q_ref):
    h = h_ref[...]
    p_ref[...] = jnp.dot(h, wd_ref[...], preferred_element_type=jnp.float32)
    q_ref[...] = jnp.dot(h, ws_ref[...], preferred_element_type=jnp.float32)


def _readout_prep(h, Wsrc, Wdst):
    return pl.pallas_call(
        _readout_prep_body,
        out_shape=(
            jax.ShapeDtypeStruct((NP, C), jnp.float32),
            jax.ShapeDtypeStruct((NP, C), jnp.float32),
        ),
    )(h, Wsrc, Wdst)


def _edge_terms_body(eas_ref, ea_ref, wme_ref, bm_ref, wre_ref, br_ref, et_ref, rt_ref):
    et_ref[...] = jnp.dot(eas_ref[...], wme_ref[...], preferred_element_type=jnp.float32) + bm_ref[...]
    rt_ref[...] = jnp.dot(ea_ref[...], wre_ref[...], preferred_element_type=jnp.float32) + br_ref[...]


def _edge_terms(eas8, ea8, Wme8, bm1, Wre8, br1, eb=4096):
    grid = EP // eb
    return pl.pallas_call(
        _edge_terms_body,
        grid=(grid,),
        in_specs=[
            pl.BlockSpec((eb, 8), lambda i: (i, 0)),
            pl.BlockSpec((eb, 8), lambda i: (i, 0)),
            pl.BlockSpec((8, C), lambda i: (0, 0)),
            pl.BlockSpec((1, C), lambda i: (0, 0)),
            pl.BlockSpec((8, C), lambda i: (0, 0)),
            pl.BlockSpec((1, C), lambda i: (0, 0)),
        ],
        out_specs=(
            pl.BlockSpec((eb, C), lambda i: (i, 0)),
            pl.BlockSpec((eb, C), lambda i: (i, 0)),
        ),
        out_shape=(
            jax.ShapeDtypeStruct((EP, C), jnp.float32),
            jax.ShapeDtypeStruct((EP, C), jnp.float32),
        ),
    )(eas8, ea8, Wme8, bm1.reshape(1, C), Wre8, br1.reshape(1, C))


def _h0_body(x_ref, w_ref, b_ref, h_ref):
    h_ref[...] = jnp.dot(x_ref[...], w_ref[...], preferred_element_type=jnp.float32) + b_ref[...]


def _h0(x128, W0128, b0):
    return pl.pallas_call(
        _h0_body,
        out_shape=jax.ShapeDtypeStruct((NP, C), jnp.float32),
    )(x128, W0128, b0.reshape(1, C))


def _mm_bias_body(r_ref, w_ref, b_ref, m_ref):
    m_ref[...] = jnp.dot(r_ref[...], w_ref[...], preferred_element_type=jnp.float32) + b_ref[...]


def _edge_mlp2(r, W2, b2, eb=4096):
    grid = EP // eb
    return pl.pallas_call(
        _mm_bias_body,
        grid=(grid,),
        in_specs=[
            pl.BlockSpec((eb, C), lambda i: (i, 0)),
            pl.BlockSpec((C, C), lambda i: (0, 0)),
            pl.BlockSpec((1, C), lambda i: (0, 0)),
        ],
        out_specs=pl.BlockSpec((eb, C), lambda i: (i, 0)),
        out_shape=jax.ShapeDtypeStruct((EP, C), jnp.float32),
    )(r, W2, b2.reshape(1, C))


def _readout2_body(t_ref, w_ref, q_ref):
    q_ref[...] = jnp.dot(t_ref[...], w_ref[...], preferred_element_type=jnp.float32)


def _readout2(t, W2pad, eb=4096):
    grid = EP // eb
    return pl.pallas_call(
        _readout2_body,
        grid=(grid,),
        in_specs=[
            pl.BlockSpec((eb, C), lambda i: (i, 0)),
            pl.BlockSpec((C, 128), lambda i: (0, 0)),
        ],
        out_specs=pl.BlockSpec((eb, 128), lambda i: (i, 0)),
        out_shape=jax.ShapeDtypeStruct((EP, 128), jnp.float32),
    )(t, W2pad)


def _update_body(s_ref, q_ref, mn_ref, mx_ref, hr_ref, inv_ref, msk_ref,
                 w1_ref, b1_ref, w2_ref, b2_ref, h_ref):
    inv = inv_ref[...]
    msk = msk_ref[...]
    mean = s_ref[...] * inv
    mean_sq = q_ref[...] * inv
    var = jnp.maximum(mean_sq - mean * mean, 0.0)
    std = jnp.sqrt(var + 1e-5) * msk
    mn = mn_ref[...] * msk
    mx = mx_ref[...] * msk
    z = jnp.concatenate([std, mn, mx, mean, hr_ref[...]], axis=1)
    u = jnp.maximum(jnp.dot(z, w1_ref[...], preferred_element_type=jnp.float32) + b1_ref[...], 0.0)
    h_ref[...] = jnp.dot(u, w2_ref[...], preferred_element_type=jnp.float32) + b2_ref[...]


def _update(s, q, mn, mx, hr, inv, msk, Wu1, bu1, Wu2, bu2):
    return pl.pallas_call(
        _update_body,
        out_shape=jax.ShapeDtypeStruct((NP, C), jnp.float32),
    )(s, q, mn, mx, hr, inv, msk, Wu1, bu1.reshape(1, C), Wu2, bu2.reshape(1, C))


def kernel(x, edge_index, edge_attr, W0, b0, Wm1, bm1, Wm2, bm2, Wu1, bu1, Wu2, bu2, Wr1, br1, Wr2, br2):
    src = edge_index[0]
    dst = edge_index[1]
    src_p = jnp.pad(src, (0, EP - E)).astype(jnp.int32)
    dst_p = jnp.pad(dst, (0, EP - E), constant_values=NB - 1).astype(jnp.int32)
    ea8 = jnp.pad(edge_attr, ((0, EP - E), (0, 6)))

    Wme8 = jnp.pad(Wm1[2 * C:], ((0, 6), (0, 0)))
    Wre8 = jnp.pad(Wr1[2 * C:], ((0, 6), (0, 0)))
    x128 = jnp.pad(x, ((0, NP - N), (0, 125)))
    W0128 = jnp.pad(W0, ((0, 125), (0, 0)))

    hists = _sc_hist(dst_p)
    cnt_bins = jnp.sum(hists, axis=0)
    csum = jnp.cumsum(cnt_bins)
    g_all = jnp.concatenate([jnp.zeros((1,), jnp.int32), csum[:-1]])
    offs = (jnp.cumsum(hists, axis=0) - hists) + g_all[None, :]
    rp = jnp.full((10032,), E, jnp.int32)
    rp = rp.at[0].set(0).at[1:N + 1].set(csum[:N])
    dsts, srcs, eas8 = _sc_place(dst_p, src_p, ea8, offs)

    et, rt = _edge_terms(eas8, ea8, Wme8, bm1, Wre8, br1)
    h = _h0(x128, W0128, b0)

    cntp = cnt_bins.astype(jnp.float32)
    inv = (1.0 / jnp.clip(cntp, 1.0))[:, None]
    msk = (cntp > 0).astype(jnp.float32)[:, None]

    for _ in range(5):
        hr, A, B = _node_prep(h, Wm1[:C], Wm1[C:2 * C])
        r = _sc_gather(A, B, et, dsts, srcs)
        m = _edge_mlp2(r, Wm2, bm2)
        s, q, mn, mx = _sc_scatter(m, dsts, rp)
        s = s.reshape(NP, C)
        q = q.reshape(NP, C)
        mn = mn.reshape(NP, C)
        mx = mx.reshape(NP, C)
        h = _update(s, q, mn, mx, hr, inv, msk, Wu1, bu1, Wu2, bu2)

    P, Q = _readout_prep(h, Wr1[:C], Wr1[C:2 * C])
    t = _sc_gather(Q, P, rt, dst_p, src_p)
    W2pad = jnp.zeros((C, 128), jnp.float32).at[:, :1].set(Wr2)
    qp = _readout2(t, W2pad)
    return qp[:E, :1] + br2


# speedup vs baseline: 2.2383x; 1.0285x over previous
import functools
import jax
import jax.numpy as jnp
from jax import lax
from jax.experimental import pallas as pl
from jax.experimental.pallas import tpu as pltpu
from jax.experimental.pallas import tpu_sc as plsc


C = 64
N = 10000
NP = 10016
E = 320000
EP = 327680
NW = 32
NC = 2
EPW = EP // NW
KG = 512
NCH = EPW // KG

_sc_mesh = plsc.VectorSubcoreMesh(core_axis_name="c", subcore_axis_name="s")


KGD = 256
NCHD = EPW // KGD


def _sc_gather_body(a_hbm, b_hbm, dst_hbm, src_hbm, ag_hbm, bg_hbm,
                    dsti, srci, abuf, bbuf, gsem, osem):
    w = lax.axis_index("s") * NC + lax.axis_index("c")
    base_w = w * EPW
    icp1 = pltpu.async_copy(dst_hbm.at[pl.ds(base_w, EPW)], dsti, gsem)
    icp2 = pltpu.async_copy(src_hbm.at[pl.ds(base_w, EPW)], srci, gsem)
    icp1.wait()
    icp2.wait()

    def fire(ci):
        slot = ci % 2
        cps = []
        for j in range(KGD // 128):
            o = ci * KGD + j * 128
            cps.append(pltpu.async_copy(
                a_hbm.at[dsti.at[pl.ds(o, 128)]],
                abuf.at[slot].at[pl.ds(j * 128, 128)], gsem))
            cps.append(pltpu.async_copy(
                b_hbm.at[srci.at[pl.ds(o, 128)]],
                bbuf.at[slot].at[pl.ds(j * 128, 128)], gsem))
        return cps

    def out(ci):
        slot = ci % 2
        base = base_w + ci * KGD
        return [
            pltpu.async_copy(abuf.at[slot], ag_hbm.at[pl.ds(base, KGD)], osem),
            pltpu.async_copy(bbuf.at[slot], bg_hbm.at[pl.ds(base, KGD)], osem),
        ]

    gcps = {0: fire(0)}
    ocps = {}
    for ci in range(NCHD):
        for cp in gcps.pop(ci):
            cp.wait()
        ocps[ci] = out(ci)
        if ci - 1 in ocps:
            for cp in ocps.pop(ci - 1):
                cp.wait()
        if ci + 1 < NCHD:
            gcps[ci + 1] = fire(ci + 1)
    for cp in ocps.pop(NCHD - 1):
        cp.wait()


_sc_gather = functools.partial(
    pl.kernel,
    out_type=(
        jax.ShapeDtypeStruct((EP, C), jnp.float32),
        jax.ShapeDtypeStruct((EP, C), jnp.float32),
    ),
    mesh=_sc_mesh,
    scratch_types=[
        pltpu.VMEM((EPW,), jnp.int32),
        pltpu.VMEM((EPW,), jnp.int32),
        pltpu.VMEM((2, KGD, C), jnp.float32),
        pltpu.VMEM((2, KGD, C), jnp.float32),
        pltpu.SemaphoreType.DMA,
        pltpu.SemaphoreType.DMA,
    ],
    compiler_params=pltpu.CompilerParams(use_tc_tiling_on_sc=False, needs_layout_passes=False),
)(_sc_gather_body)


NB = 10016
NPT = 313


def _iota16():
    return lax.broadcasted_iota(jnp.int32, (16,), 0)


def _sc_hist_body(dst_hbm, out_hbm, hist, dbuf, sem):
    w = lax.axis_index("s") * NC + lax.axis_index("c")
    ones = jnp.ones((16,), jnp.int32)

    def zero(i, carry):
        hist[pl.ds(i * 16, 16)] = jnp.zeros((16,), jnp.int32)
        return carry

    lax.fori_loop(0, NB // 16, zero, 0)

    def chunk(ci, carry):
        base = w * EPW + ci * KG
        pltpu.sync_copy(dst_hbm.at[pl.ds(base, KG)], dbuf)

        def group(g, c2):
            v = dbuf[pl.ds(g * 16, 16)]
            plsc.addupdate_scatter(hist, [v], ones)
            return c2

        lax.fori_loop(0, KG // 16, group, 0)
        return carry

    lax.fori_loop(0, NCH, chunk, 0)
    pltpu.sync_copy(hist, out_hbm.at[w])


_sc_hist = functools.partial(
    pl.kernel,
    out_type=jax.ShapeDtypeStruct((NW, NB), jnp.int32),
    mesh=_sc_mesh,
    scratch_types=[
        pltpu.VMEM((NB,), jnp.int32),
        pltpu.VMEM((KG,), jnp.int32),
        pltpu.SemaphoreType.DMA,
    ],
    compiler_params=pltpu.CompilerParams(use_tc_tiling_on_sc=False, needs_layout_passes=False),
)(_sc_hist_body)


def _lane_gather(v, idx):
    return lax.gather(
        v, idx[:, None],
        dimension_numbers=lax.GatherDimensionNumbers(
            offset_dims=(), collapsed_slice_dims=(0,), start_index_map=(0,)),
        slice_sizes=(1,),
        mode=lax.GatherScatterMode.PROMISE_IN_BOUNDS)


def _dup_rank(v):
    iota = _iota16()
    rank = jnp.zeros((16,), jnp.int32)
    for k in range(1, 16):
        shifted = _lane_gather(v, jnp.clip(iota - k, 0, 15))
        eq = jnp.logical_and(shifted == v, iota >= k)
        rank = rank + eq.astype(jnp.int32)
    return rank


def _sc_place_body(dst_hbm, src_hbm, ea_hbm, offs_hbm,
                   dsts_hbm, srcs_hbm, eas_hbm,
                   ctr, dstb, srcb, eab, posb, sem):
    w = lax.axis_index("s") * NC + lax.axis_index("c")
    ones = jnp.ones((16,), jnp.int32)
    pltpu.sync_copy(offs_hbm.at[w], ctr)

    def chunk(ci, carry):
        base = w * EPW + ci * KG
        for j in range(4):
            pltpu.sync_copy(dst_hbm.at[pl.ds(base + j * 128, 128)], dstb.at[j])
            pltpu.sync_copy(src_hbm.at[pl.ds(base + j * 128, 128)], srcb.at[j])
            pltpu.sync_copy(ea_hbm.at[pl.ds(base + j * 128, 128)], eab.at[j])
        for j in range(4):
            for g in range(8):
                v = dstb[j, pl.ds(g * 16, 16)]
                old = plsc.load_gather(ctr, [v])
                rank = _dup_rank(v)
                plsc.addupdate_scatter(ctr, [v], ones)
                posb[j, pl.ds(g * 16, 16)] = old + rank
        cps = []
        for j in range(4):
            cps.append(pltpu.async_copy(dstb.at[j], dsts_hbm.at[posb.at[j]], sem))
            cps.append(pltpu.async_copy(srcb.at[j], srcs_hbm.at[posb.at[j]], sem))
            cps.append(pltpu.async_copy(eab.at[j], eas_hbm.at[posb.at[j]], sem))
        for cp in cps:
            cp.wait()
        return carry

    lax.fori_loop(0, NCH, chunk, 0)


_sc_place = functools.partial(
    pl.kernel,
    out_type=(
        jax.ShapeDtypeStruct((EP,), jnp.int32),
        jax.ShapeDtypeStruct((EP,), jnp.int32),
        jax.ShapeDtypeStruct((EP, 8), jnp.float32),
    ),
    mesh=_sc_mesh,
    scratch_types=[
        pltpu.VMEM((NB,), jnp.int32),
        pltpu.VMEM((4, 128), jnp.int32),
        pltpu.VMEM((4, 128), jnp.int32),
        pltpu.VMEM((4, 128, 8), jnp.float32),
        pltpu.VMEM((4, 128), jnp.int32),
        pltpu.SemaphoreType.DMA,
    ],
    compiler_params=pltpu.CompilerParams(use_tc_tiling_on_sc=False, needs_layout_passes=False),
)(_sc_place_body)


K2 = 256
ACC = NPT * C


def _sc_scatter_body(m_hbm, dsts_hbm, rp_hbm,
                     aggs_hbm, aggq_hbm, aggn_hbm, aggx_hbm,
                     accs, accq, accn, accx, mbuf, dbuf, rpb, sem):
    w = lax.axis_index("s") * NC + lax.axis_index("c")
    nbase = w * NPT
    iota = _iota16()

    def fetch(idx):
        b16 = (idx // 16) * 16
        pltpu.sync_copy(rp_hbm.at[pl.ds(b16, 16)], rpb)
        v = rpb[...]
        return jnp.sum(jnp.where(iota == (idx - b16), v, 0))

    start = fetch(nbase)
    end = fetch(nbase + NPT)

    def zero(i, carry):
        sl = pl.ds(i * 16, 16)
        accs[sl] = jnp.zeros((16,), jnp.float32)
        accq[sl] = jnp.zeros((16,), jnp.float32)
        accn[sl] = jnp.full((16,), 1e30, jnp.float32)
        accx[sl] = jnp.full((16,), -1e30, jnp.float32)
        return carry

    lax.fori_loop(0, ACC // 16, zero, 0)

    start16 = (start // 16) * 16
    nch = (end - start16 + K2 - 1) // K2

    def chunk(ci, carry):
        base = start16 + ci * K2
        pltpu.sync_copy(m_hbm.at[pl.ds(base, K2)], mbuf)
        pltpu.sync_copy(dsts_hbm.at[pl.ds(base, K2)], dbuf)

        def group(g, c2):
            dv = dbuf[pl.ds(g * 16, 16)]
            locc = jnp.clip(dv - nbase, 0, NPT - 1)
            for e in range(16):
                sel = iota == e
                row = jnp.sum(jnp.where(sel, locc, 0))
                pos = base + g * 16 + e
                vald = jnp.where(
                    jnp.logical_and(pos >= start, pos < end), 1, 0)
                vb = jnp.full((16,), vald, jnp.int32) > 0
                vf = jnp.where(vb, 1.0, 0.0)
                off = row * C
                for grp in range(4):
                    mv = mbuf[g * 16 + e, pl.ds(grp * 16, 16)]
                    sl = pl.ds(off + grp * 16, 16)
                    accs[sl] = accs[sl] + mv * vf
                    accq[sl] = accq[sl] + mv * mv * vf
                    accn[sl] = jnp.minimum(accn[sl], jnp.where(vb, mv, 1e30))
                    accx[sl] = jnp.maximum(accx[sl], jnp.where(vb, mv, -1e30))
            return c2

        lax.fori_loop(0, K2 // 16, group, 0)
        return carry

    lax.fori_loop(0, nch, chunk, 0)
    obase = w * ACC
    pltpu.sync_copy(accs, aggs_hbm.at[pl.ds(obase, ACC)])
    pltpu.sync_copy(accq, aggq_hbm.at[pl.ds(obase, ACC)])
    pltpu.sync_copy(accn, aggn_hbm.at[pl.ds(obase, ACC)])
    pltpu.sync_copy(accx, aggx_hbm.at[pl.ds(obase, ACC)])


_sc_scatter = functools.partial(
    pl.kernel,
    out_type=(
        jax.ShapeDtypeStruct((NP * C,), jnp.float32),
        jax.ShapeDtypeStruct((NP * C,), jnp.float32),
        jax.ShapeDtypeStruct((NP * C,), jnp.float32),
        jax.ShapeDtypeStruct((NP * C,), jnp.float32),
    ),
    mesh=_sc_mesh,
    scratch_types=[
        pltpu.VMEM((ACC,), jnp.float32),
        pltpu.VMEM((ACC,), jnp.float32),
        pltpu.VMEM((ACC,), jnp.float32),
        pltpu.VMEM((ACC,), jnp.float32),
        pltpu.VMEM((K2, C), jnp.float32),
        pltpu.VMEM((K2,), jnp.int32),
        pltpu.VMEM((16,), jnp.int32),
        pltpu.SemaphoreType.DMA,
    ],
    compiler_params=pltpu.CompilerParams(use_tc_tiling_on_sc=False, needs_layout_passes=False),
)(_sc_scatter_body)


def _node_prep_body(h_ref, wd_ref, ws_ref, hr_ref, a_ref, b_ref):
    hr = jnp.maximum(h_ref[...], 0.0)
    hr_ref[...] = hr
    a_ref[...] = jnp.dot(hr, wd_ref[...], preferred_element_type=jnp.float32)
    b_ref[...] = jnp.dot(hr, ws_ref[...], preferred_element_type=jnp.float32)


def _node_prep(h, Wd, Ws):
    return pl.pallas_call(
        _node_prep_body,
        out_shape=(
            jax.ShapeDtypeStruct((NP, C), jnp.float32),
            jax.ShapeDtypeStruct((NP, C), jnp.float32),
            jax.ShapeDtypeStruct((NP, C), jnp.float32),
        ),
    )(h, Wd, Ws)


def _readout_prep_body(h_ref, wd_ref, ws_ref, p_ref, q_ref):
    h = h_ref[...]
    p_ref[...] = jnp.dot(h, wd_ref[...], preferred_element_type=jnp.float32)
    q_ref[...] = jnp.dot(h, ws_ref[...], preferred_element_type=jnp.float32)


def _readout_prep(h, Wsrc, Wdst):
    return pl.pallas_call(
        _readout_prep_body,
        out_shape=(
            jax.ShapeDtypeStruct((NP, C), jnp.float32),
            jax.ShapeDtypeStruct((NP, C), jnp.float32),
        ),
    )(h, Wsrc, Wdst)


def _edge_terms_body(eas_ref, ea_ref, wme_ref, bm_ref, wre_ref, br_ref, et_ref, rt_ref):
    et_ref[...] = jnp.dot(eas_ref[...], wme_ref[...], preferred_element_type=jnp.float32) + bm_ref[...]
    rt_ref[...] = jnp.dot(ea_ref[...], wre_ref[...], preferred_element_type=jnp.float32) + br_ref[...]


def _edge_terms(eas8, ea8, Wme8, bm1, Wre8, br1, eb=4096):
    grid = EP // eb
    return pl.pallas_call(
        _edge_terms_body,
        grid=(grid,),
        in_specs=[
            pl.BlockSpec((eb, 8), lambda i: (i, 0)),
            pl.BlockSpec((eb, 8), lambda i: (i, 0)),
            pl.BlockSpec((8, C), lambda i: (0, 0)),
            pl.BlockSpec((1, C), lambda i: (0, 0)),
            pl.BlockSpec((8, C), lambda i: (0, 0)),
            pl.BlockSpec((1, C), lambda i: (0, 0)),
        ],
        out_specs=(
            pl.BlockSpec((eb, C), lambda i: (i, 0)),
            pl.BlockSpec((eb, C), lambda i: (i, 0)),
        ),
        out_shape=(
            jax.ShapeDtypeStruct((EP, C), jnp.float32),
            jax.ShapeDtypeStruct((EP, C), jnp.float32),
        ),
    )(eas8, ea8, Wme8, bm1.reshape(1, C), Wre8, br1.reshape(1, C))


def _h0_body(x_ref, w_ref, b_ref, h_ref):
    h_ref[...] = jnp.dot(x_ref[...], w_ref[...], preferred_element_type=jnp.float32) + b_ref[...]


def _h0(x128, W0128, b0):
    return pl.pallas_call(
        _h0_body,
        out_shape=jax.ShapeDtypeStruct((NP, C), jnp.float32),
    )(x128, W0128, b0.reshape(1, C))


def _mm_bias_body(ag_ref, bg_ref, et_ref, w_ref, b_ref, m_ref):
    r = jnp.maximum(ag_ref[...] + bg_ref[...] + et_ref[...], 0.0)
    m_ref[...] = jnp.dot(r, w_ref[...], preferred_element_type=jnp.float32) + b_ref[...]


def _edge_mlp2(ag, bg, et, W2, b2, eb=4096):
    grid = EP // eb
    return pl.pallas_call(
        _mm_bias_body,
        grid=(grid,),
        in_specs=[
            pl.BlockSpec((eb, C), lambda i: (i, 0)),
            pl.BlockSpec((eb, C), lambda i: (i, 0)),
            pl.BlockSpec((eb, C), lambda i: (i, 0)),
            pl.BlockSpec((C, C), lambda i: (0, 0)),
            pl.BlockSpec((1, C), lambda i: (0, 0)),
        ],
        out_specs=pl.BlockSpec((eb, C), lambda i: (i, 0)),
        out_shape=jax.ShapeDtypeStruct((EP, C), jnp.float32),
    )(ag, bg, et, W2, b2.reshape(1, C))


def _readout2_body(ag_ref, bg_ref, rt_ref, w_ref, q_ref):
    t = jnp.maximum(ag_ref[...] + bg_ref[...] + rt_ref[...], 0.0)
    q_ref[...] = jnp.dot(t, w_ref[...], preferred_element_type=jnp.float32)


def _readout2(ag, bg, rt, W2pad, eb=4096):
    grid = EP // eb
    return pl.pallas_call(
        _readout2_body,
        grid=(grid,),
        in_specs=[
            pl.BlockSpec((eb, C), lambda i: (i, 0)),
            pl.BlockSpec((eb, C), lambda i: (i, 0)),
            pl.BlockSpec((eb, C), lambda i: (i, 0)),
            pl.BlockSpec((C, 128), lambda i: (0, 0)),
        ],
        out_specs=pl.BlockSpec((eb, 128), lambda i: (i, 0)),
        out_shape=jax.ShapeDtypeStruct((EP, 128), jnp.float32),
    )(ag, bg, rt, W2pad)


def _update_body(s_ref, q_ref, mn_ref, mx_ref, hr_ref, inv_ref, msk_ref,
                 w1_ref, b1_ref, w2_ref, b2_ref, h_ref):
    inv = inv_ref[...]
    msk = msk_ref[...]
    mean = s_ref[...] * inv
    mean_sq = q_ref[...] * inv
    var = jnp.maximum(mean_sq - mean * mean, 0.0)
    std = jnp.sqrt(var + 1e-5) * msk
    mn = mn_ref[...] * msk
    mx = mx_ref[...] * msk
    z = jnp.concatenate([std, mn, mx, mean, hr_ref[...]], axis=1)
    u = jnp.maximum(jnp.dot(z, w1_ref[...], preferred_element_type=jnp.float32) + b1_ref[...], 0.0)
    h_ref[...] = jnp.dot(u, w2_ref[...], preferred_element_type=jnp.float32) + b2_ref[...]


def _update(s, q, mn, mx, hr, inv, msk, Wu1, bu1, Wu2, bu2):
    return pl.pallas_call(
        _update_body,
        out_shape=jax.ShapeDtypeStruct((NP, C), jnp.float32),
    )(s, q, mn, mx, hr, inv, msk, Wu1, bu1.reshape(1, C), Wu2, bu2.reshape(1, C))


def kernel(x, edge_index, edge_attr, W0, b0, Wm1, bm1, Wm2, bm2, Wu1, bu1, Wu2, bu2, Wr1, br1, Wr2, br2):
    src = edge_index[0]
    dst = edge_index[1]
    src_p = jnp.pad(src, (0, EP - E)).astype(jnp.int32)
    dst_p = jnp.pad(dst, (0, EP - E), constant_values=NB - 1).astype(jnp.int32)
    ea8 = jnp.pad(edge_attr, ((0, EP - E), (0, 6)))

    Wme8 = jnp.pad(Wm1[2 * C:], ((0, 6), (0, 0)))
    Wre8 = jnp.pad(Wr1[2 * C:], ((0, 6), (0, 0)))
    x128 = jnp.pad(x, ((0, NP - N), (0, 125)))
    W0128 = jnp.pad(W0, ((0, 125), (0, 0)))

    hists = _sc_hist(dst_p)
    cnt_bins = jnp.sum(hists, axis=0)
    csum = jnp.cumsum(cnt_bins)
    g_all = jnp.concatenate([jnp.zeros((1,), jnp.int32), csum[:-1]])
    offs = (jnp.cumsum(hists, axis=0) - hists) + g_all[None, :]
    rp = jnp.full((10032,), E, jnp.int32)
    rp = rp.at[0].set(0).at[1:N + 1].set(csum[:N])
    dsts, srcs, eas8 = _sc_place(dst_p, src_p, ea8, offs)

    et, rt = _edge_terms(eas8, ea8, Wme8, bm1, Wre8, br1)
    h = _h0(x128, W0128, b0)

    cntp = cnt_bins.astype(jnp.float32)
    inv = (1.0 / jnp.clip(cntp, 1.0))[:, None]
    msk = (cntp > 0).astype(jnp.float32)[:, None]

    for _ in range(5):
        hr, A, B = _node_prep(h, Wm1[:C], Wm1[C:2 * C])
        ag, bg = _sc_gather(A, B, dsts, srcs)
        m = _edge_mlp2(ag, bg, et, Wm2, bm2)
        s, q, mn, mx = _sc_scatter(m, dsts, rp)
        s = s.reshape(NP, C)
        q = q.reshape(NP, C)
        mn = mn.reshape(NP, C)
        mx = mx.reshape(NP, C)
        h = _update(s, q, mn, mx, hr, inv, msk, Wu1, bu1, Wu2, bu2)

    P, Q = _readout_prep(h, Wr1[:C], Wr1[C:2 * C])
    qg, pg = _sc_gather(Q, P, dst_p, src_p)
    W2pad = jnp.zeros((C, 128), jnp.float32).at[:, :1].set(Wr2)
    qp = _readout2(qg, pg, rt, W2pad)
    return qp[:E, :1] + br2


# speedup vs baseline: 2.3327x; 1.0422x over previous
import functools
import jax
import jax.numpy as jnp
from jax import lax
from jax.experimental import pallas as pl
from jax.experimental.pallas import tpu as pltpu
from jax.experimental.pallas import tpu_sc as plsc


C = 64
N = 10000
NP = 10016
E = 320000
EP = 327680
NW = 32
NC = 2
EPW = EP // NW
KG = 512
NCH = EPW // KG

_sc_mesh = plsc.VectorSubcoreMesh(core_axis_name="c", subcore_axis_name="s")


KGD = 128
NCHD = EPW // KGD
NSLOT = 6


def _sc_gather_body(a_hbm, b_hbm, dst_hbm, src_hbm, ag_hbm, bg_hbm,
                    dsti, srci, abuf, bbuf, gsem, osem):
    w = lax.axis_index("s") * NC + lax.axis_index("c")
    base_w = w * EPW
    icp1 = pltpu.async_copy(dst_hbm.at[pl.ds(base_w, EPW)], dsti, gsem)
    icp2 = pltpu.async_copy(src_hbm.at[pl.ds(base_w, EPW)], srci, gsem)
    icp1.wait()
    icp2.wait()

    def fire(ci):
        slot = ci % NSLOT
        o = ci * KGD
        return [
            pltpu.async_copy(a_hbm.at[dsti.at[pl.ds(o, KGD)]],
                             abuf.at[slot], gsem),
            pltpu.async_copy(b_hbm.at[srci.at[pl.ds(o, KGD)]],
                             bbuf.at[slot], gsem),
        ]

    def out(ci):
        slot = ci % NSLOT
        base = base_w + ci * KGD
        return [
            pltpu.async_copy(abuf.at[slot], ag_hbm.at[pl.ds(base, KGD)], osem),
            pltpu.async_copy(bbuf.at[slot], bg_hbm.at[pl.ds(base, KGD)], osem),
        ]

    gcps = {}
    ocps = {}
    for ci in range(min(NSLOT, NCHD)):
        gcps[ci] = fire(ci)
    for ci in range(NCHD):
        for cp in gcps.pop(ci):
            cp.wait()
        ocps[ci] = out(ci)
        nxt = ci + NSLOT
        if nxt < NCHD:
            for cp in ocps.pop(ci):
                cp.wait()
            gcps[nxt] = fire(nxt)
    for rem in sorted(ocps):
        for cp in ocps.pop(rem):
            cp.wait()


_sc_gather = functools.partial(
    pl.kernel,
    out_type=(
        jax.ShapeDtypeStruct((EP, C), jnp.float32),
        jax.ShapeDtypeStruct((EP, C), jnp.float32),
    ),
    mesh=_sc_mesh,
    scratch_types=[
        pltpu.VMEM((EPW,), jnp.int32),
        pltpu.VMEM((EPW,), jnp.int32),
        pltpu.VMEM((NSLOT, KGD, C), jnp.float32),
        pltpu.VMEM((NSLOT, KGD, C), jnp.float32),
        pltpu.SemaphoreType.DMA,
        pltpu.SemaphoreType.DMA,
    ],
    compiler_params=pltpu.CompilerParams(use_tc_tiling_on_sc=False, needs_layout_passes=False),
)(_sc_gather_body)


NB = 10016
NPT = 313


def _iota16():
    return lax.broadcasted_iota(jnp.int32, (16,), 0)


def _sc_hist_body(dst_hbm, out_hbm, hist, dbuf, sem):
    w = lax.axis_index("s") * NC + lax.axis_index("c")
    ones = jnp.ones((16,), jnp.int32)

    def zero(i, carry):
        hist[pl.ds(i * 16, 16)] = jnp.zeros((16,), jnp.int32)
        return carry

    lax.fori_loop(0, NB // 16, zero, 0)

    def chunk(ci, carry):
        base = w * EPW + ci * KG
        pltpu.sync_copy(dst_hbm.at[pl.ds(base, KG)], dbuf)

        def group(g, c2):
            v = dbuf[pl.ds(g * 16, 16)]
            plsc.addupdate_scatter(hist, [v], ones)
            return c2

        lax.fori_loop(0, KG // 16, group, 0)
        return carry

    lax.fori_loop(0, NCH, chunk, 0)
    pltpu.sync_copy(hist, out_hbm.at[w])


_sc_hist = functools.partial(
    pl.kernel,
    out_type=jax.ShapeDtypeStruct((NW, NB), jnp.int32),
    mesh=_sc_mesh,
    scratch_types=[
        pltpu.VMEM((NB,), jnp.int32),
        pltpu.VMEM((KG,), jnp.int32),
        pltpu.SemaphoreType.DMA,
    ],
    compiler_params=pltpu.CompilerParams(use_tc_tiling_on_sc=False, needs_layout_passes=False),
)(_sc_hist_body)


def _lane_gather(v, idx):
    return lax.gather(
        v, idx[:, None],
        dimension_numbers=lax.GatherDimensionNumbers(
            offset_dims=(), collapsed_slice_dims=(0,), start_index_map=(0,)),
        slice_sizes=(1,),
        mode=lax.GatherScatterMode.PROMISE_IN_BOUNDS)


def _dup_rank(v):
    iota = _iota16()
    rank = jnp.zeros((16,), jnp.int32)
    for k in range(1, 16):
        shifted = _lane_gather(v, jnp.clip(iota - k, 0, 15))
        eq = jnp.logical_and(shifted == v, iota >= k)
        rank = rank + eq.astype(jnp.int32)
    return rank


def _sc_place_body(dst_hbm, src_hbm, ea_hbm, offs_hbm,
                   dsts_hbm, srcs_hbm, eas_hbm,
                   ctr, dstb, srcb, eab, posb, sem):
    w = lax.axis_index("s") * NC + lax.axis_index("c")
    ones = jnp.ones((16,), jnp.int32)
    pltpu.sync_copy(offs_hbm.at[w], ctr)

    def chunk(ci, carry):
        base = w * EPW + ci * KG
        for j in range(4):
            pltpu.sync_copy(dst_hbm.at[pl.ds(base + j * 128, 128)], dstb.at[j])
            pltpu.sync_copy(src_hbm.at[pl.ds(base + j * 128, 128)], srcb.at[j])
            pltpu.sync_copy(ea_hbm.at[pl.ds(base + j * 128, 128)], eab.at[j])
        for j in range(4):
            for g in range(8):
                v = dstb[j, pl.ds(g * 16, 16)]
                old = plsc.load_gather(ctr, [v])
                rank = _dup_rank(v)
                plsc.addupdate_scatter(ctr, [v], ones)
                posb[j, pl.ds(g * 16, 16)] = old + rank
        cps = []
        for j in range(4):
            cps.append(pltpu.async_copy(dstb.at[j], dsts_hbm.at[posb.at[j]], sem))
            cps.append(pltpu.async_copy(srcb.at[j], srcs_hbm.at[posb.at[j]], sem))
            cps.append(pltpu.async_copy(eab.at[j], eas_hbm.at[posb.at[j]], sem))
        for cp in cps:
            cp.wait()
        return carry

    lax.fori_loop(0, NCH, chunk, 0)


_sc_place = functools.partial(
    pl.kernel,
    out_type=(
        jax.ShapeDtypeStruct((EP,), jnp.int32),
        jax.ShapeDtypeStruct((EP,), jnp.int32),
        jax.ShapeDtypeStruct((EP, 8), jnp.float32),
    ),
    mesh=_sc_mesh,
    scratch_types=[
        pltpu.VMEM((NB,), jnp.int32),
        pltpu.VMEM((4, 128), jnp.int32),
        pltpu.VMEM((4, 128), jnp.int32),
        pltpu.VMEM((4, 128, 8), jnp.float32),
        pltpu.VMEM((4, 128), jnp.int32),
        pltpu.SemaphoreType.DMA,
    ],
    compiler_params=pltpu.CompilerParams(use_tc_tiling_on_sc=False, needs_layout_passes=False),
)(_sc_place_body)


K2 = 256
ACC = NPT * C


def _sc_scatter_body(m_hbm, dsts_hbm, rp_hbm,
                     aggs_hbm, aggq_hbm, aggn_hbm, aggx_hbm,
                     accs, accq, accn, accx, mbuf, dbuf, rpb, sem):
    w = lax.axis_index("s") * NC + lax.axis_index("c")
    nbase = w * NPT
    iota = _iota16()

    def fetch(idx):
        b16 = (idx // 16) * 16
        pltpu.sync_copy(rp_hbm.at[pl.ds(b16, 16)], rpb)
        v = rpb[...]
        return jnp.sum(jnp.where(iota == (idx - b16), v, 0))

    start = fetch(nbase)
    end = fetch(nbase + NPT)

    def zero(i, carry):
        sl = pl.ds(i * 16, 16)
        accs[sl] = jnp.zeros((16,), jnp.float32)
        accq[sl] = jnp.zeros((16,), jnp.float32)
        accn[sl] = jnp.full((16,), 1e30, jnp.float32)
        accx[sl] = jnp.full((16,), -1e30, jnp.float32)
        return carry

    lax.fori_loop(0, ACC // 16, zero, 0)

    start16 = (start // 16) * 16
    nch = (end - start16 + K2 - 1) // K2

    def chunk(ci, carry):
        base = start16 + ci * K2
        pltpu.sync_copy(m_hbm.at[pl.ds(base, K2)], mbuf)
        pltpu.sync_copy(dsts_hbm.at[pl.ds(base, K2)], dbuf)

        def group(g, c2):
            dv = dbuf[pl.ds(g * 16, 16)]
            locc = jnp.clip(dv - nbase, 0, NPT - 1)
            for e in range(16):
                sel = iota == e
                row = jnp.sum(jnp.where(sel, locc, 0))
                pos = base + g * 16 + e
                vald = jnp.where(
                    jnp.logical_and(pos >= start, pos < end), 1, 0)
                vb = jnp.full((16,), vald, jnp.int32) > 0
                vf = jnp.where(vb, 1.0, 0.0)
                off = row * C
                for grp in range(4):
                    mv = mbuf[g * 16 + e, pl.ds(grp * 16, 16)]
                    sl = pl.ds(off + grp * 16, 16)
                    accs[sl] = accs[sl] + mv * vf
                    accq[sl] = accq[sl] + mv * mv * vf
                    accn[sl] = jnp.minimum(accn[sl], jnp.where(vb, mv, 1e30))
                    accx[sl] = jnp.maximum(accx[sl], jnp.where(vb, mv, -1e30))
            return c2

        lax.fori_loop(0, K2 // 16, group, 0)
        return carry

    lax.fori_loop(0, nch, chunk, 0)
    obase = w * ACC
    pltpu.sync_copy(accs, aggs_hbm.at[pl.ds(obase, ACC)])
    pltpu.sync_copy(accq, aggq_hbm.at[pl.ds(obase, ACC)])
    pltpu.sync_copy(accn, aggn_hbm.at[pl.ds(obase, ACC)])
    pltpu.sync_copy(accx, aggx_hbm.at[pl.ds(obase, ACC)])


_sc_scatter = functools.partial(
    pl.kernel,
    out_type=(
        jax.ShapeDtypeStruct((NP * C,), jnp.float32),
        jax.ShapeDtypeStruct((NP * C,), jnp.float32),
        jax.ShapeDtypeStruct((NP * C,), jnp.float32),
        jax.ShapeDtypeStruct((NP * C,), jnp.float32),
    ),
    mesh=_sc_mesh,
    scratch_types=[
        pltpu.VMEM((ACC,), jnp.float32),
        pltpu.VMEM((ACC,), jnp.float32),
        pltpu.VMEM((ACC,), jnp.float32),
        pltpu.VMEM((ACC,), jnp.float32),
        pltpu.VMEM((K2, C), jnp.float32),
        pltpu.VMEM((K2,), jnp.int32),
        pltpu.VMEM((16,), jnp.int32),
        pltpu.SemaphoreType.DMA,
    ],
    compiler_params=pltpu.CompilerParams(use_tc_tiling_on_sc=False, needs_layout_passes=False),
)(_sc_scatter_body)


def _node_prep_body(h_ref, wd_ref, ws_ref, hr_ref, a_ref, b_ref):
    hr = jnp.maximum(h_ref[...], 0.0)
    hr_ref[...] = hr
    a_ref[...] = jnp.dot(hr, wd_ref[...], preferred_element_type=jnp.float32)
    b_ref[...] = jnp.dot(hr, ws_ref[...], preferred_element_type=jnp.float32)


def _node_prep(h, Wd, Ws):
    return pl.pallas_call(
        _node_prep_body,
        out_shape=(
            jax.ShapeDtypeStruct((NP, C), jnp.float32),
            jax.ShapeDtypeStruct((NP, C), jnp.float32),
            jax.ShapeDtypeStruct((NP, C), jnp.float32),
        ),
    )(h, Wd, Ws)


def _readout_prep_body(h_ref, wd_ref, ws_ref, p_ref, q_ref):
    h = h_ref[...]
    p_ref[...] = jnp.dot(h, wd_ref[...], preferred_element_type=jnp.float32)
    q_ref[...] = jnp.dot(h, ws_ref[...], preferred_element_type=jnp.float32)


def _readout_prep(h, Wsrc, Wdst):
    return pl.pallas_call(
        _readout_prep_body,
        out_shape=(
            jax.ShapeDtypeStruct((NP, C), jnp.float32),
            jax.ShapeDtypeStruct((NP, C), jnp.float32),
        ),
    )(h, Wsrc, Wdst)


def _edge_terms_body(eas_ref, ea_ref, wme_ref, bm_ref, wre_ref, br_ref, et_ref, rt_ref):
    et_ref[...] = jnp.dot(eas_ref[...], wme_ref[...], preferred_element_type=jnp.float32) + bm_ref[...]
    rt_ref[...] = jnp.dot(ea_ref[...], wre_ref[...], preferred_element_type=jnp.float32) + br_ref[...]


def _edge_terms(eas8, ea8, Wme8, bm1, Wre8, br1, eb=4096):
    grid = EP // eb
    return pl.pallas_call(
        _edge_terms_body,
        grid=(grid,),
        in_specs=[
            pl.BlockSpec((eb, 8), lambda i: (i, 0)),
            pl.BlockSpec((eb, 8), lambda i: (i, 0)),
            pl.BlockSpec((8, C), lambda i: (0, 0)),
            pl.BlockSpec((1, C), lambda i: (0, 0)),
            pl.BlockSpec((8, C), lambda i: (0, 0)),
            pl.BlockSpec((1, C), lambda i: (0, 0)),
        ],
        out_specs=(
            pl.BlockSpec((eb, C), lambda i: (i, 0)),
            pl.BlockSpec((eb, C), lambda i: (i, 0)),
        ),
        out_shape=(
            jax.ShapeDtypeStruct((EP, C), jnp.float32),
            jax.ShapeDtypeStruct((EP, C), jnp.float32),
        ),
    )(eas8, ea8, Wme8, bm1.reshape(1, C), Wre8, br1.reshape(1, C))


def _h0_body(x_ref, w_ref, b_ref, h_ref):
    h_ref[...] = jnp.dot(x_ref[...], w_ref[...], preferred_element_type=jnp.float32) + b_ref[...]


def _h0(x128, W0128, b0):
    return pl.pallas_call(
        _h0_body,
        out_shape=jax.ShapeDtypeStruct((NP, C), jnp.float32),
    )(x128, W0128, b0.reshape(1, C))


def _mm_bias_body(ag_ref, bg_ref, et_ref, w_ref, b_ref, m_ref):
    r = jnp.maximum(ag_ref[...] + bg_ref[...] + et_ref[...], 0.0)
    m_ref[...] = jnp.dot(r, w_ref[...], preferred_element_type=jnp.float32) + b_ref[...]


def _edge_mlp2(ag, bg, et, W2, b2, eb=4096):
    grid = EP // eb
    return pl.pallas_call(
        _mm_bias_body,
        grid=(grid,),
        in_specs=[
            pl.BlockSpec((eb, C), lambda i: (i, 0)),
            pl.BlockSpec((eb, C), lambda i: (i, 0)),
            pl.BlockSpec((eb, C), lambda i: (i, 0)),
            pl.BlockSpec((C, C), lambda i: (0, 0)),
            pl.BlockSpec((1, C), lambda i: (0, 0)),
        ],
        out_specs=pl.BlockSpec((eb, C), lambda i: (i, 0)),
        out_shape=jax.ShapeDtypeStruct((EP, C), jnp.float32),
    )(ag, bg, et, W2, b2.reshape(1, C))


def _readout2_body(ag_ref, bg_ref, rt_ref, w_ref, q_ref):
    t = jnp.maximum(ag_ref[...] + bg_ref[...] + rt_ref[...], 0.0)
    q_ref[...] = jnp.dot(t, w_ref[...], preferred_element_type=jnp.float32)


def _readout2(ag, bg, rt, W2pad, eb=4096):
    grid = EP // eb
    return pl.pallas_call(
        _readout2_body,
        grid=(grid,),
        in_specs=[
            pl.BlockSpec((eb, C), lambda i: (i, 0)),
            pl.BlockSpec((eb, C), lambda i: (i, 0)),
            pl.BlockSpec((eb, C), lambda i: (i, 0)),
            pl.BlockSpec((C, 128), lambda i: (0, 0)),
        ],
        out_specs=pl.BlockSpec((eb, 128), lambda i: (i, 0)),
        out_shape=jax.ShapeDtypeStruct((EP, 128), jnp.float32),
    )(ag, bg, rt, W2pad)


def _update_body(s_ref, q_ref, mn_ref, mx_ref, hr_ref, inv_ref, msk_ref,
                 w1_ref, b1_ref, w2_ref, b2_ref, h_ref):
    inv = inv_ref[...]
    msk = msk_ref[...]
    mean = s_ref[...] * inv
    mean_sq = q_ref[...] * inv
    var = jnp.maximum(mean_sq - mean * mean, 0.0)
    std = jnp.sqrt(var + 1e-5) * msk
    mn = mn_ref[...] * msk
    mx = mx_ref[...] * msk
    z = jnp.concatenate([std, mn, mx, mean, hr_ref[...]], axis=1)
    u = jnp.maximum(jnp.dot(z, w1_ref[...], preferred_element_type=jnp.float32) + b1_ref[...], 0.0)
    h_ref[...] = jnp.dot(u, w2_ref[...], preferred_element_type=jnp.float32) + b2_ref[...]


def _update(s, q, mn, mx, hr, inv, msk, Wu1, bu1, Wu2, bu2):
    return pl.pallas_call(
        _update_body,
        out_shape=jax.ShapeDtypeStruct((NP, C), jnp.float32),
    )(s, q, mn, mx, hr, inv, msk, Wu1, bu1.reshape(1, C), Wu2, bu2.reshape(1, C))


def kernel(x, edge_index, edge_attr, W0, b0, Wm1, bm1, Wm2, bm2, Wu1, bu1, Wu2, bu2, Wr1, br1, Wr2, br2):
    src = edge_index[0]
    dst = edge_index[1]
    src_p = jnp.pad(src, (0, EP - E)).astype(jnp.int32)
    dst_p = jnp.pad(dst, (0, EP - E), constant_values=NB - 1).astype(jnp.int32)
    ea8 = jnp.pad(edge_attr, ((0, EP - E), (0, 6)))

    Wme8 = jnp.pad(Wm1[2 * C:], ((0, 6), (0, 0)))
    Wre8 = jnp.pad(Wr1[2 * C:], ((0, 6), (0, 0)))
    x128 = jnp.pad(x, ((0, NP - N), (0, 125)))
    W0128 = jnp.pad(W0, ((0, 125), (0, 0)))

    hists = _sc_hist(dst_p)
    cnt_bins = jnp.sum(hists, axis=0)
    csum = jnp.cumsum(cnt_bins)
    g_all = jnp.concatenate([jnp.zeros((1,), jnp.int32), csum[:-1]])
    offs = (jnp.cumsum(hists, axis=0) - hists) + g_all[None, :]
    rp = jnp.full((10032,), E, jnp.int32)
    rp = rp.at[0].set(0).at[1:N + 1].set(csum[:N])
    dsts, srcs, eas8 = _sc_place(dst_p, src_p, ea8, offs)

    et, rt = _edge_terms(eas8, ea8, Wme8, bm1, Wre8, br1)
    h = _h0(x128, W0128, b0)

    cntp = cnt_bins.astype(jnp.float32)
    inv = (1.0 / jnp.clip(cntp, 1.0))[:, None]
    msk = (cntp > 0).astype(jnp.float32)[:, None]

    for _ in range(5):
        hr, A, B = _node_prep(h, Wm1[:C], Wm1[C:2 * C])
        ag, bg = _sc_gather(A, B, dsts, srcs)
        m = _edge_mlp2(ag, bg, et, Wm2, bm2)
        s, q, mn, mx = _sc_scatter(m, dsts, rp)
        s = s.reshape(NP, C)
        q = q.reshape(NP, C)
        mn = mn.reshape(NP, C)
        mx = mx.reshape(NP, C)
        h = _update(s, q, mn, mx, hr, inv, msk, Wu1, bu1, Wu2, bu2)

    P, Q = _readout_prep(h, Wr1[:C], Wr1[C:2 * C])
    qg, pg = _sc_gather(Q, P, dst_p, src_p)
    W2pad = jnp.zeros((C, 128), jnp.float32).at[:, :1].set(Wr2)
    qp = _readout2(qg, pg, rt, W2pad)
    return qp[:E, :1] + br2
